# baseline (speedup 1.0000x reference)
import functools
import math

import jax
import jax.numpy as jnp
from jax import lax
from jax.experimental import pallas as pl
from jax.experimental.pallas import tpu as pltpu
from jax.experimental.pallas import tpu_sc as plsc

_N = 10000
_IN = 128
_H = 64
_OUT = 128
_E = 320000
_BN_EPS = 1e-5

_NC = 2
_NS = 16
_NW = _NC * _NS
_CH = 128
_NCHUNK = 2560
_EP = _NCHUNK * _CH
_K = _NCHUNK // _NW
_SPARE = 640
_RPT = 624
_RLAST = _N - 15 * _RPT
_AGG_ROWS = _N + _SPARE
_ZPT = _AGG_ROWS // _NS

_BM = 1000


_NPOS = _E // 2 // _CH
_NREAL = _E // _CH


def _sc_scatter_partials(table, src3, tgt3):
    mesh = plsc.VectorSubcoreMesh(core_axis_name="c", subcore_axis_name="s")
    nbuf = 4

    @functools.partial(
        pl.kernel,
        out_type=jax.ShapeDtypeStruct((_NC, _N, 2 * _H), jnp.float32),
        mesh=mesh,
        scratch_types=[
            pltpu.VMEM((_K, _CH), jnp.int32),
            pltpu.VMEM((_K, _CH), jnp.int32),
            pltpu.VMEM((nbuf, _CH, _H), jnp.float32),
            pltpu.VMEM((340, _H), jnp.float32),
            pltpu.VMEM_SHARED((_AGG_ROWS, _H), jnp.float32),
            [pltpu.SemaphoreType.DMA] * nbuf,
            [pltpu.SemaphoreType.DMA] * nbuf,
            pltpu.SemaphoreType.DMA,
            pltpu.SemaphoreType.DMA,
        ],
        compiler_params=pltpu.CompilerParams(use_tc_tiling_on_sc=False),
    )
    def run(table_hbm, src_hbm, tgt_hbm, out_hbm,
            src_v, tgt_v, rows, zbuf, agg, sem_g, sem_s, sem_i1, sem_i2):
        c = lax.axis_index("c")
        s = lax.axis_index("s")
        base = (c * _NS + s) * _K

        di_s = pltpu.async_copy(src_hbm.at[pl.ds(base, _K)], src_v, sem_i1)
        di_t = pltpu.async_copy(tgt_hbm.at[pl.ds(base, _K)], tgt_v, sem_i2)

        zero = jnp.zeros((16,), jnp.float32)

        def zrow(i, carry):
            for q in range(_H // 16):
                zbuf[i, pl.ds(q * 16, 16)] = zero
            return carry

        lax.fori_loop(0, 340, zrow, 0)

        pltpu.sync_copy(zbuf, agg.at[pl.ds(s * _ZPT, 340)])
        pltpu.sync_copy(zbuf.at[pl.ds(0, _ZPT - 340)],
                        agg.at[pl.ds(s * _ZPT + 340, _ZPT - 340)])

        plsc.subcore_barrier()
        di_s.wait()
        di_t.wait()

        def quad(m, carry):
            j0 = nbuf * m

            @pl.when(m > 0)
            def _():
                for t in range(nbuf):
                    pltpu.make_async_copy(rows.at[t], agg.at[tgt_v.at[j0 + t]],
                                          sem_s[t]).wait()

            gs = [pltpu.async_copy(table_hbm.at[src_v.at[j0 + t]],
                                   rows.at[t], sem_g[t])
                  for t in range(nbuf)]
            for t in range(nbuf):
                gs[t].wait()
                pltpu.async_copy(rows.at[t], agg.at[tgt_v.at[j0 + t]],
                                 sem_s[t], add=True)
            return carry

        lax.fori_loop(0, _K // nbuf, quad, 0)
        for t in range(nbuf):
            pltpu.make_async_copy(rows.at[t], agg.at[tgt_v.at[t]],
                                  sem_s[t]).wait()
        plsc.subcore_barrier()

        @pl.when(s < _NS - 1)
        def _():
            pltpu.sync_copy(agg.at[pl.ds(s * _RPT, _RPT)],
                            out_hbm.at[c, pl.ds(s * _RPT, _RPT), pl.ds(0, _H)])

        @pl.when(s == _NS - 1)
        def _():
            pltpu.sync_copy(agg.at[pl.ds(15 * _RPT, _RLAST)],
                            out_hbm.at[c, pl.ds(15 * _RPT, _RLAST), pl.ds(0, _H)])

    return run(table, src3, tgt3)


def _tc_mm3(x, w0, b0, w1, b1, ws, bs, combine=None):
    d_in = w0.shape[0]
    wcat = jnp.concatenate([w0, w1], axis=1)
    bcat = jnp.concatenate([b0, b1]).reshape(1, 2 * _H)

    def body(*refs):
        if combine is not None:
            p_ref, ys_ref, wc_ref, bc_ref, ws_ref, bs_ref, tab_ref, yso_ref = refs
            xb = jnp.tanh(p_ref[0][:, :_H] + p_ref[1][:, :_H] + ys_ref[...])
        else:
            x_ref, wc_ref, bc_ref, ws_ref, bs_ref, tab_ref, yso_ref = refs
            xb = x_ref[...]
        tab_ref[...] = jnp.dot(xb, wc_ref[...], preferred_element_type=jnp.float32) + bc_ref[...]
        yso_ref[...] = jnp.dot(xb, ws_ref[...], preferred_element_type=jnp.float32) + bs_ref[...]

    if combine is not None:
        parts, ys = combine
        lead_args = (parts, ys)
        lead_specs = [pl.BlockSpec((2, _BM, 2 * _H), lambda i: (0, i, 0)),
                      pl.BlockSpec((_BM, _H), lambda i: (i, 0))]
    else:
        lead_args = (x,)
        lead_specs = [pl.BlockSpec((_BM, d_in), lambda i: (i, 0))]

    return pl.pallas_call(
        body,
        grid=(_N // _BM,),
        in_specs=lead_specs + [pl.BlockSpec((d_in, 2 * _H), lambda i: (0, 0)),
                               pl.BlockSpec((1, 2 * _H), lambda i: (0, 0)),
                               pl.BlockSpec((d_in, _H), lambda i: (0, 0)),
                               pl.BlockSpec((1, _H), lambda i: (0, 0))],
        out_specs=[pl.BlockSpec((_BM, 2 * _H), lambda i: (i, 0)),
                   pl.BlockSpec((_BM, _H), lambda i: (i, 0))],
        out_shape=[jax.ShapeDtypeStruct((_N, 2 * _H), jnp.float32),
                   jax.ShapeDtypeStruct((_N, _H), jnp.float32)],
    )(*lead_args, wcat, bcat, ws, bs.reshape(1, _H))


def _tc_final(parts, ys, ww, wb, mw1, mb1, g1, be1, mw2, mb2, g2, be2, w3row, b3):
    bn_s = 1.0 / math.sqrt(1.0 + _BN_EPS)
    wb, mb1, g1, be1, mb2, g2, be2 = (a.reshape(1, _OUT) for a in
                                      (wb, mb1, g1, be1, mb2, g2, be2))
    b3 = b3.reshape(1, 1)

    def body(p_ref, ys_ref, ww_ref, wb_ref, mw1_ref, mb1_ref, g1_ref, be1_ref,
             mw2_ref, mb2_ref, g2_ref, be2_ref, w3_ref, b3_ref, z_ref, prob_ref):
        z2 = jnp.tanh(p_ref[0][:, :_H] + p_ref[1][:, :_H] + ys_ref[...])
        z = jnp.tanh(jnp.dot(z2, ww_ref[...], preferred_element_type=jnp.float32) + wb_ref[...])
        z_ref[...] = z
        h = jnp.dot(z, mw1_ref[...], preferred_element_type=jnp.float32) + mb1_ref[...]
        h = jnp.maximum(h * bn_s * g1_ref[...] + be1_ref[...], 0.0)
        h = jnp.dot(h, mw2_ref[...], preferred_element_type=jnp.float32) + mb2_ref[...]
        h = jnp.maximum(h * bn_s * g2_ref[...] + be2_ref[...], 0.0)
        logit = jnp.sum(h * w3_ref[...], axis=1, keepdims=True) + b3_ref[...]
        prob_ref[...] = jax.nn.sigmoid(logit)

    full = lambda a: pl.BlockSpec(a.shape, lambda i: tuple(0 for _ in a.shape))
    return pl.pallas_call(
        body,
        grid=(_N // _BM,),
        in_specs=[pl.BlockSpec((2, _BM, 2 * _H), lambda i: (0, i, 0)),
                  pl.BlockSpec((_BM, _H), lambda i: (i, 0)),
                  full(ww), full(wb), full(mw1), full(mb1), full(g1), full(be1),
                  full(mw2), full(mb2), full(g2), full(be2), full(w3row), full(b3)],
        out_specs=[pl.BlockSpec((_BM, _OUT), lambda i: (i, 0)),
                   pl.BlockSpec((_BM, 1), lambda i: (i, 0))],
        out_shape=[jax.ShapeDtypeStruct((_N, _OUT), jnp.float32),
                   jax.ShapeDtypeStruct((_N, 1), jnp.float32)],
    )(parts, ys, ww, wb, mw1, mb1, g1, be1, mw2, mb2, g2, be2, w3row, b3)


def kernel(init_emb, edge_index_s, params):
    p = params
    pos = edge_index_s[:, : _E // 2]
    neg = edge_index_s[:, _E // 2:]
    pad = _EP - _E
    ar = jnp.arange(pad, dtype=jnp.int32)
    src = jnp.concatenate([pos[0] * 2, neg[0] * 2 + 1, (ar % _N) * 2])
    tgt = jnp.concatenate([pos[1], neg[1], _N + (ar % _SPARE)])
    src3 = src.reshape(_NCHUNK, _CH)
    tgt3 = tgt.reshape(_NCHUNK, _CH)
    tab1, ys1 = _tc_mm3(init_emb, p['c1_w0'], p['c1_b0'], p['c1_w1'], p['c1_b1'],
                        p['c1_ws'], p['c1_bs'])
    part1 = _sc_scatter_partials(tab1.reshape(2 * _N, _H), src3, tgt3)
    tab2, ys2 = _tc_mm3(None, p['c2_w0'], p['c2_b0'], p['c2_w1'], p['c2_b1'],
                        p['c2_ws'], p['c2_bs'], combine=(part1, ys1))
    part2 = _sc_scatter_partials(tab2.reshape(2 * _N, _H), src3, tgt3)
    z, prob = _tc_final(part2, ys2, p['w_w'], p['w_b'],
                        p['m_w1'], p['m_b1'], p['m_g1'], p['m_be1'],
                        p['m_w2'], p['m_b2'], p['m_g2'], p['m_be2'],
                        p['m_w3'].reshape(1, _OUT), p['m_b3'])
    return (z, prob)


# speedup vs baseline: 1.0314x; 1.0314x over previous
import functools
import math

import jax
import jax.numpy as jnp
from jax import lax
from jax.experimental import pallas as pl
from jax.experimental.pallas import tpu as pltpu
from jax.experimental.pallas import tpu_sc as plsc

_N = 10000
_IN = 128
_H = 64
_OUT = 128
_E = 320000
_BN_EPS = 1e-5

_NC = 2
_NS = 16
_NW = _NC * _NS
_CH = 128
_NCHUNK = 2560
_EP = _NCHUNK * _CH
_K = _NCHUNK // _NW
_SPARE = 640
_RPT = 624
_RLAST = _N - 15 * _RPT
_AGG_ROWS = _N + _SPARE
_ZPT = _AGG_ROWS // _NS

_BM = 1000


_NPOS = _E // 2 // _CH
_NREAL = _E // _CH


def _sc_scatter_partials(table, src3, tgt3):
    mesh = plsc.VectorSubcoreMesh(core_axis_name="c", subcore_axis_name="s")
    nbuf = 5

    @functools.partial(
        pl.kernel,
        out_type=jax.ShapeDtypeStruct((_NC, _N, 2 * _H), jnp.float32),
        mesh=mesh,
        scratch_types=[
            pltpu.VMEM((_K, _CH), jnp.int32),
            pltpu.VMEM((_K, _CH), jnp.int32),
            pltpu.VMEM((nbuf, _CH, _H), jnp.float32),
            pltpu.VMEM((340, _H), jnp.float32),
            pltpu.VMEM_SHARED((_AGG_ROWS, _H), jnp.float32),
            [pltpu.SemaphoreType.DMA] * nbuf,
            [pltpu.SemaphoreType.DMA] * nbuf,
            pltpu.SemaphoreType.DMA,
            pltpu.SemaphoreType.DMA,
        ],
        compiler_params=pltpu.CompilerParams(use_tc_tiling_on_sc=False),
    )
    def run(table_hbm, src_hbm, tgt_hbm, out_hbm,
            src_v, tgt_v, rows, zbuf, agg, sem_g, sem_s, sem_i1, sem_i2):
        c = lax.axis_index("c")
        s = lax.axis_index("s")
        base = (c * _NS + s) * _K

        di_s = pltpu.async_copy(src_hbm.at[pl.ds(base, _K)], src_v, sem_i1)
        di_t = pltpu.async_copy(tgt_hbm.at[pl.ds(base, _K)], tgt_v, sem_i2)

        zero = jnp.zeros((16,), jnp.float32)

        def zrow(i, carry):
            for q in range(_H // 16):
                zbuf[i, pl.ds(q * 16, 16)] = zero
            return carry

        lax.fori_loop(0, 340, zrow, 0)

        pltpu.sync_copy(zbuf, agg.at[pl.ds(s * _ZPT, 340)])
        pltpu.sync_copy(zbuf.at[pl.ds(0, _ZPT - 340)],
                        agg.at[pl.ds(s * _ZPT + 340, _ZPT - 340)])

        plsc.subcore_barrier()
        di_s.wait()
        di_t.wait()

        def quad(m, carry):
            j0 = nbuf * m

            @pl.when(m > 0)
            def _():
                for t in range(nbuf):
                    pltpu.make_async_copy(rows.at[t], agg.at[tgt_v.at[j0 + t]],
                                          sem_s[t]).wait()

            gs = [pltpu.async_copy(table_hbm.at[src_v.at[j0 + t]],
                                   rows.at[t], sem_g[t])
                  for t in range(nbuf)]
            for t in range(nbuf):
                gs[t].wait()
                pltpu.async_copy(rows.at[t], agg.at[tgt_v.at[j0 + t]],
                                 sem_s[t], add=True)
            return carry

        lax.fori_loop(0, _K // nbuf, quad, 0)
        for t in range(nbuf):
            pltpu.make_async_copy(rows.at[t], agg.at[tgt_v.at[t]],
                                  sem_s[t]).wait()
        plsc.subcore_barrier()

        @pl.when(s < _NS - 1)
        def _():
            pltpu.sync_copy(agg.at[pl.ds(s * _RPT, _RPT)],
                            out_hbm.at[c, pl.ds(s * _RPT, _RPT), pl.ds(0, _H)])

        @pl.when(s == _NS - 1)
        def _():
            pltpu.sync_copy(agg.at[pl.ds(15 * _RPT, _RLAST)],
                            out_hbm.at[c, pl.ds(15 * _RPT, _RLAST), pl.ds(0, _H)])

    return run(table, src3, tgt3)


def _tc_mm3(x, w0, b0, w1, b1, ws, bs, combine=None):
    d_in = w0.shape[0]
    wcat = jnp.concatenate([w0, w1], axis=1)
    bcat = jnp.concatenate([b0, b1]).reshape(1, 2 * _H)

    def body(*refs):
        if combine is not None:
            p_ref, ys_ref, wc_ref, bc_ref, ws_ref, bs_ref, tab_ref, yso_ref = refs
            xb = jnp.tanh(p_ref[0][:, :_H] + p_ref[1][:, :_H] + ys_ref[...])
        else:
            x_ref, wc_ref, bc_ref, ws_ref, bs_ref, tab_ref, yso_ref = refs
            xb = x_ref[...]
        tab_ref[...] = jnp.dot(xb, wc_ref[...], preferred_element_type=jnp.float32) + bc_ref[...]
        yso_ref[...] = jnp.dot(xb, ws_ref[...], preferred_element_type=jnp.float32) + bs_ref[...]

    if combine is not None:
        parts, ys = combine
        lead_args = (parts, ys)
        lead_specs = [pl.BlockSpec((2, _BM, 2 * _H), lambda i: (0, i, 0)),
                      pl.BlockSpec((_BM, _H), lambda i: (i, 0))]
    else:
        lead_args = (x,)
        lead_specs = [pl.BlockSpec((_BM, d_in), lambda i: (i, 0))]

    return pl.pallas_call(
        body,
        grid=(_N // _BM,),
        in_specs=lead_specs + [pl.BlockSpec((d_in, 2 * _H), lambda i: (0, 0)),
                               pl.BlockSpec((1, 2 * _H), lambda i: (0, 0)),
                               pl.BlockSpec((d_in, _H), lambda i: (0, 0)),
                               pl.BlockSpec((1, _H), lambda i: (0, 0))],
        out_specs=[pl.BlockSpec((_BM, 2 * _H), lambda i: (i, 0)),
                   pl.BlockSpec((_BM, _H), lambda i: (i, 0))],
        out_shape=[jax.ShapeDtypeStruct((_N, 2 * _H), jnp.float32),
                   jax.ShapeDtypeStruct((_N, _H), jnp.float32)],
    )(*lead_args, wcat, bcat, ws, bs.reshape(1, _H))


def _tc_final(parts, ys, ww, wb, mw1, mb1, g1, be1, mw2, mb2, g2, be2, w3row, b3):
    bn_s = 1.0 / math.sqrt(1.0 + _BN_EPS)
    wb, mb1, g1, be1, mb2, g2, be2 = (a.reshape(1, _OUT) for a in
                                      (wb, mb1, g1, be1, mb2, g2, be2))
    b3 = b3.reshape(1, 1)

    def body(p_ref, ys_ref, ww_ref, wb_ref, mw1_ref, mb1_ref, g1_ref, be1_ref,
             mw2_ref, mb2_ref, g2_ref, be2_ref, w3_ref, b3_ref, z_ref, prob_ref):
        z2 = jnp.tanh(p_ref[0][:, :_H] + p_ref[1][:, :_H] + ys_ref[...])
        z = jnp.tanh(jnp.dot(z2, ww_ref[...], preferred_element_type=jnp.float32) + wb_ref[...])
        z_ref[...] = z
        h = jnp.dot(z, mw1_ref[...], preferred_element_type=jnp.float32) + mb1_ref[...]
        h = jnp.maximum(h * bn_s * g1_ref[...] + be1_ref[...], 0.0)
        h = jnp.dot(h, mw2_ref[...], preferred_element_type=jnp.float32) + mb2_ref[...]
        h = jnp.maximum(h * bn_s * g2_ref[...] + be2_ref[...], 0.0)
        logit = jnp.sum(h * w3_ref[...], axis=1, keepdims=True) + b3_ref[...]
        prob_ref[...] = jax.nn.sigmoid(logit)

    full = lambda a: pl.BlockSpec(a.shape, lambda i: tuple(0 for _ in a.shape))
    return pl.pallas_call(
        body,
        grid=(_N // _BM,),
        in_specs=[pl.BlockSpec((2, _BM, 2 * _H), lambda i: (0, i, 0)),
                  pl.BlockSpec((_BM, _H), lambda i: (i, 0)),
                  full(ww), full(wb), full(mw1), full(mb1), full(g1), full(be1),
                  full(mw2), full(mb2), full(g2), full(be2), full(w3row), full(b3)],
        out_specs=[pl.BlockSpec((_BM, _OUT), lambda i: (i, 0)),
                   pl.BlockSpec((_BM, 1), lambda i: (i, 0))],
        out_shape=[jax.ShapeDtypeStruct((_N, _OUT), jnp.float32),
                   jax.ShapeDtypeStruct((_N, 1), jnp.float32)],
    )(parts, ys, ww, wb, mw1, mb1, g1, be1, mw2, mb2, g2, be2, w3row, b3)


def kernel(init_emb, edge_index_s, params):
    p = params
    pos = edge_index_s[:, : _E // 2]
    neg = edge_index_s[:, _E // 2:]
    pad = _EP - _E
    ar = jnp.arange(pad, dtype=jnp.int32)
    src = jnp.concatenate([pos[0] * 2, neg[0] * 2 + 1, (ar % _N) * 2])
    tgt = jnp.concatenate([pos[1], neg[1], _N + (ar % _SPARE)])
    src3 = src.reshape(_NCHUNK, _CH)
    tgt3 = tgt.reshape(_NCHUNK, _CH)
    tab1, ys1 = _tc_mm3(init_emb, p['c1_w0'], p['c1_b0'], p['c1_w1'], p['c1_b1'],
                        p['c1_ws'], p['c1_bs'])
    part1 = _sc_scatter_partials(tab1.reshape(2 * _N, _H), src3, tgt3)
    tab2, ys2 = _tc_mm3(None, p['c2_w0'], p['c2_b0'], p['c2_w1'], p['c2_b1'],
                        p['c2_ws'], p['c2_bs'], combine=(part1, ys1))
    part2 = _sc_scatter_partials(tab2.reshape(2 * _N, _H), src3, tgt3)
    z, prob = _tc_final(part2, ys2, p['w_w'], p['w_b'],
                        p['m_w1'], p['m_b1'], p['m_g1'], p['m_be1'],
                        p['m_w2'], p['m_b2'], p['m_g2'], p['m_be2'],
                        p['m_w3'].reshape(1, _OUT), p['m_b3'])
    return (z, prob)


# speedup vs baseline: 1.0454x; 1.0136x over previous
import functools
import math

import jax
import jax.numpy as jnp
from jax import lax
from jax.experimental import pallas as pl
from jax.experimental.pallas import tpu as pltpu
from jax.experimental.pallas import tpu_sc as plsc

_N = 10000
_IN = 128
_H = 64
_OUT = 128
_E = 320000
_BN_EPS = 1e-5

_NC = 2
_NS = 16
_NW = _NC * _NS
_CH = 128
_NCHUNK = 2560
_EP = _NCHUNK * _CH
_K = _NCHUNK // _NW
_SPARE = 640
_RPT = 624
_RLAST = _N - 15 * _RPT
_AGG_ROWS = _N + _SPARE
_ZPT = _AGG_ROWS // _NS

_BM = 1000


_NPOS = _E // 2 // _CH
_NREAL = _E // _CH
_KTAIL = _NREAL - (_NW - 1) * _K


def _sc_scatter_partials(table, ei3):
    mesh = plsc.VectorSubcoreMesh(core_axis_name="c", subcore_axis_name="s")
    nbuf = 5

    @functools.partial(
        pl.kernel,
        out_type=jax.ShapeDtypeStruct((_NC, _N, 2 * _H), jnp.float32),
        mesh=mesh,
        scratch_types=[
            pltpu.VMEM((_K, _CH), jnp.int32),
            pltpu.VMEM((_K, _CH), jnp.int32),
            pltpu.VMEM((nbuf, _CH, _H), jnp.float32),
            pltpu.VMEM((340, _H), jnp.float32),
            pltpu.VMEM_SHARED((_AGG_ROWS, _H), jnp.float32),
            [pltpu.SemaphoreType.DMA] * nbuf,
            [pltpu.SemaphoreType.DMA] * nbuf,
            pltpu.SemaphoreType.DMA,
            pltpu.SemaphoreType.DMA,
        ],
        compiler_params=pltpu.CompilerParams(use_tc_tiling_on_sc=False),
    )
    def run(table_hbm, ei_hbm, out_hbm,
            src_v, tgt_v, rows, zbuf, agg, sem_g, sem_s, sem_i1, sem_i2):
        c = lax.axis_index("c")
        s = lax.axis_index("s")
        tid = c * _NS + s
        base = tid * _K

        @pl.when(tid < _NW - 1)
        def _():
            pltpu.async_copy(ei_hbm.at[0, pl.ds(base, _K)], src_v, sem_i1)
            pltpu.async_copy(ei_hbm.at[1, pl.ds(base, _K)], tgt_v, sem_i2)

        @pl.when(tid == _NW - 1)
        def _():
            pltpu.async_copy(ei_hbm.at[0, pl.ds(base, _KTAIL)],
                             src_v.at[pl.ds(0, _KTAIL)], sem_i1)
            pltpu.async_copy(ei_hbm.at[1, pl.ds(base, _KTAIL)],
                             tgt_v.at[pl.ds(0, _KTAIL)], sem_i2)

        zero = jnp.zeros((16,), jnp.float32)

        def zrow(i, carry):
            for q in range(_H // 16):
                zbuf[i, pl.ds(q * 16, 16)] = zero
            return carry

        lax.fori_loop(0, 340, zrow, 0)

        pltpu.sync_copy(zbuf, agg.at[pl.ds(s * _ZPT, 340)])
        pltpu.sync_copy(zbuf.at[pl.ds(0, _ZPT - 340)],
                        agg.at[pl.ds(s * _ZPT + 340, _ZPT - 340)])

        @pl.when(tid < _NW - 1)
        def _():
            pltpu.make_async_copy(ei_hbm.at[0, pl.ds(base, _K)],
                                  src_v, sem_i1).wait()
            pltpu.make_async_copy(ei_hbm.at[1, pl.ds(base, _K)],
                                  tgt_v, sem_i2).wait()

        @pl.when(tid == _NW - 1)
        def _():
            pltpu.make_async_copy(ei_hbm.at[0, pl.ds(base, _KTAIL)],
                                  src_v.at[pl.ds(0, _KTAIL)], sem_i1).wait()
            pltpu.make_async_copy(ei_hbm.at[1, pl.ds(base, _KTAIL)],
                                  tgt_v.at[pl.ds(0, _KTAIL)], sem_i2).wait()

        def trow(k, carry):
            g = base + k
            rel = jnp.where(g >= _NPOS, 1, 0).astype(jnp.int32)

            @pl.when(g < _NREAL)
            def _():
                for q in range(_CH // 16):
                    v = src_v[k, pl.ds(q * 16, 16)]
                    src_v[k, pl.ds(q * 16, 16)] = v + v + rel
            return carry

        lax.fori_loop(0, _K, trow, 0)

        lane = lax.iota(jnp.int32, 16)

        @pl.when(tid == _NW - 1)
        def _():
            def prow(k, r):
                for q in range(_CH // 16):
                    slot = (k - _KTAIL) * _CH + q * 16 + lane
                    src_v[k, pl.ds(q * 16, 16)] = slot * 2
                    tgt_v[k, pl.ds(q * 16, 16)] = _N + r * _CH + q * 16 + lane
                return jnp.where(r == 4, 0, r + 1)

            lax.fori_loop(_KTAIL, _K, prow, 0)

        plsc.subcore_barrier()

        def quad(m, carry):
            j0 = nbuf * m

            @pl.when(m > 0)
            def _():
                for t in range(nbuf):
                    pltpu.make_async_copy(rows.at[t], agg.at[tgt_v.at[j0 + t]],
                                          sem_s[t]).wait()

            gs = [pltpu.async_copy(table_hbm.at[src_v.at[j0 + t]],
                                   rows.at[t], sem_g[t])
                  for t in range(nbuf)]
            for t in range(nbuf):
                gs[t].wait()
                pltpu.async_copy(rows.at[t], agg.at[tgt_v.at[j0 + t]],
                                 sem_s[t], add=True)
            return carry

        lax.fori_loop(0, _K // nbuf, quad, 0)
        for t in range(nbuf):
            pltpu.make_async_copy(rows.at[t], agg.at[tgt_v.at[t]],
                                  sem_s[t]).wait()
        plsc.subcore_barrier()

        @pl.when(s < _NS - 1)
        def _():
            pltpu.sync_copy(agg.at[pl.ds(s * _RPT, _RPT)],
                            out_hbm.at[c, pl.ds(s * _RPT, _RPT), pl.ds(0, _H)])

        @pl.when(s == _NS - 1)
        def _():
            pltpu.sync_copy(agg.at[pl.ds(15 * _RPT, _RLAST)],
                            out_hbm.at[c, pl.ds(15 * _RPT, _RLAST), pl.ds(0, _H)])

    return run(table, ei3)


def _tc_mm3(x, w0, b0, w1, b1, ws, bs, combine=None):
    d_in = w0.shape[0]
    wcat = jnp.concatenate([w0, w1], axis=1)
    bcat = jnp.concatenate([b0, b1]).reshape(1, 2 * _H)

    def body(*refs):
        if combine is not None:
            p_ref, ys_ref, wc_ref, bc_ref, ws_ref, bs_ref, tab_ref, yso_ref = refs
            xb = jnp.tanh(p_ref[0][:, :_H] + p_ref[1][:, :_H] + ys_ref[...])
        else:
            x_ref, wc_ref, bc_ref, ws_ref, bs_ref, tab_ref, yso_ref = refs
            xb = x_ref[...]
        tab_ref[...] = jnp.dot(xb, wc_ref[...], preferred_element_type=jnp.float32) + bc_ref[...]
        yso_ref[...] = jnp.dot(xb, ws_ref[...], preferred_element_type=jnp.float32) + bs_ref[...]

    if combine is not None:
        parts, ys = combine
        lead_args = (parts, ys)
        lead_specs = [pl.BlockSpec((2, _BM, 2 * _H), lambda i: (0, i, 0)),
                      pl.BlockSpec((_BM, _H), lambda i: (i, 0))]
    else:
        lead_args = (x,)
        lead_specs = [pl.BlockSpec((_BM, d_in), lambda i: (i, 0))]

    return pl.pallas_call(
        body,
        grid=(_N // _BM,),
        in_specs=lead_specs + [pl.BlockSpec((d_in, 2 * _H), lambda i: (0, 0)),
                               pl.BlockSpec((1, 2 * _H), lambda i: (0, 0)),
                               pl.BlockSpec((d_in, _H), lambda i: (0, 0)),
                               pl.BlockSpec((1, _H), lambda i: (0, 0))],
        out_specs=[pl.BlockSpec((_BM, 2 * _H), lambda i: (i, 0)),
                   pl.BlockSpec((_BM, _H), lambda i: (i, 0))],
        out_shape=[jax.ShapeDtypeStruct((_N, 2 * _H), jnp.float32),
                   jax.ShapeDtypeStruct((_N, _H), jnp.float32)],
    )(*lead_args, wcat, bcat, ws, bs.reshape(1, _H))


def _tc_final(parts, ys, ww, wb, mw1, mb1, g1, be1, mw2, mb2, g2, be2, w3row, b3):
    bn_s = 1.0 / math.sqrt(1.0 + _BN_EPS)
    wb, mb1, g1, be1, mb2, g2, be2 = (a.reshape(1, _OUT) for a in
                                      (wb, mb1, g1, be1, mb2, g2, be2))
    b3 = b3.reshape(1, 1)

    def body(p_ref, ys_ref, ww_ref, wb_ref, mw1_ref, mb1_ref, g1_ref, be1_ref,
             mw2_ref, mb2_ref, g2_ref, be2_ref, w3_ref, b3_ref, z_ref, prob_ref):
        z2 = jnp.tanh(p_ref[0][:, :_H] + p_ref[1][:, :_H] + ys_ref[...])
        z = jnp.tanh(jnp.dot(z2, ww_ref[...], preferred_element_type=jnp.float32) + wb_ref[...])
        z_ref[...] = z
        h = jnp.dot(z, mw1_ref[...], preferred_element_type=jnp.float32) + mb1_ref[...]
        h = jnp.maximum(h * bn_s * g1_ref[...] + be1_ref[...], 0.0)
        h = jnp.dot(h, mw2_ref[...], preferred_element_type=jnp.float32) + mb2_ref[...]
        h = jnp.maximum(h * bn_s * g2_ref[...] + be2_ref[...], 0.0)
        logit = jnp.sum(h * w3_ref[...], axis=1, keepdims=True) + b3_ref[...]
        prob_ref[...] = jax.nn.sigmoid(logit)

    full = lambda a: pl.BlockSpec(a.shape, lambda i: tuple(0 for _ in a.shape))
    return pl.pallas_call(
        body,
        grid=(_N // _BM,),
        in_specs=[pl.BlockSpec((2, _BM, 2 * _H), lambda i: (0, i, 0)),
                  pl.BlockSpec((_BM, _H), lambda i: (i, 0)),
                  full(ww), full(wb), full(mw1), full(mb1), full(g1), full(be1),
                  full(mw2), full(mb2), full(g2), full(be2), full(w3row), full(b3)],
        out_specs=[pl.BlockSpec((_BM, _OUT), lambda i: (i, 0)),
                   pl.BlockSpec((_BM, 1), lambda i: (i, 0))],
        out_shape=[jax.ShapeDtypeStruct((_N, _OUT), jnp.float32),
                   jax.ShapeDtypeStruct((_N, 1), jnp.float32)],
    )(parts, ys, ww, wb, mw1, mb1, g1, be1, mw2, mb2, g2, be2, w3row, b3)


def kernel(init_emb, edge_index_s, params):
    p = params
    ei3 = edge_index_s.reshape(2, _NREAL, _CH)
    tab1, ys1 = _tc_mm3(init_emb, p['c1_w0'], p['c1_b0'], p['c1_w1'], p['c1_b1'],
                        p['c1_ws'], p['c1_bs'])
    part1 = _sc_scatter_partials(tab1.reshape(2 * _N, _H), ei3)
    tab2, ys2 = _tc_mm3(None, p['c2_w0'], p['c2_b0'], p['c2_w1'], p['c2_b1'],
                        p['c2_ws'], p['c2_bs'], combine=(part1, ys1))
    part2 = _sc_scatter_partials(tab2.reshape(2 * _N, _H), ei3)
    z, prob = _tc_final(part2, ys2, p['w_w'], p['w_b'],
                        p['m_w1'], p['m_b1'], p['m_g1'], p['m_be1'],
                        p['m_w2'], p['m_b2'], p['m_g2'], p['m_be2'],
                        p['m_w3'].reshape(1, _OUT), p['m_b3'])
    return (z, prob)
